# Initial kernel scaffold; baseline (speedup 1.0000x reference)
import functools

import jax
import jax.numpy as jnp
from jax.experimental import pallas as pl
from jax.experimental.pallas import tpu as pltpu

_K = 16
_PAD_VAL = 1.0e18
_BIG_IDX = jnp.int32(2**30)


def _knn_kernel(q_ref, r_ref, d_out, i_out, best_d, best_i, *, C, NC, prec):
    nc = pl.program_id(1)

    q = q_ref[0]
    r = r_ref[0]
    q2 = jnp.sum(q * q, axis=1)
    r2 = jnp.sum(r * r, axis=1)
    qr = jax.lax.dot_general(
        q, r, (((1,), (1,)), ((), ())),
        preferred_element_type=jnp.float32, precision=prec)
    d2 = q2[:, None] + r2[None, :] - 2.0 * qr
    d2 = jnp.maximum(d2, 0.0)

    M = q.shape[0]
    gidx = nc * C + jax.lax.broadcasted_iota(jnp.int32, (M, C), 1)

    @pl.when(nc == 0)
    def _init():
        best_d[...] = jnp.full((M, _K), jnp.inf, jnp.float32)
        best_i[...] = _BIG_IDX + jax.lax.broadcasted_iota(jnp.int32, (M, _K), 1)

    v = jnp.concatenate([best_d[...], d2], axis=1)
    g = jnp.concatenate([best_i[...], gidx], axis=1)

    mins, sels = [], []
    for _ in range(_K):
        mn = jnp.min(v, axis=1, keepdims=True)
        cand = jnp.where(v == mn, g, jnp.int32(2**31 - 1))
        sel = jnp.min(cand, axis=1, keepdims=True)
        mins.append(mn)
        sels.append(sel)
        v = jnp.where(g == sel, jnp.inf, v)
    best_d[...] = jnp.concatenate(mins, axis=1)
    best_i[...] = jnp.concatenate(sels, axis=1)

    @pl.when(nc == NC - 1)
    def _fin():
        d_out[0] = jnp.sqrt(best_d[...])
        i_out[0] = best_i[...]


def _knn_topk(ref, query, C, prec):
    B, N, D = ref.shape
    M = query.shape[1]
    NC = -(-N // C)
    Npad = NC * C
    if Npad != N:
        ref = jnp.pad(ref, ((0, 0), (0, Npad - N), (0, 0)),
                      constant_values=_PAD_VAL)

    grid = (B, NC)
    kern = functools.partial(_knn_kernel, C=C, NC=NC, prec=prec)
    d, i = pl.pallas_call(
        kern,
        grid=grid,
        in_specs=[
            pl.BlockSpec((1, M, D), lambda b, n: (b, 0, 0)),
            pl.BlockSpec((1, C, D), lambda b, n: (b, n, 0)),
        ],
        out_specs=[
            pl.BlockSpec((1, M, _K), lambda b, n: (b, 0, 0)),
            pl.BlockSpec((1, M, _K), lambda b, n: (b, 0, 0)),
        ],
        out_shape=[
            jax.ShapeDtypeStruct((B, M, _K), jnp.float32),
            jax.ShapeDtypeStruct((B, M, _K), jnp.int32),
        ],
        scratch_shapes=[
            pltpu.VMEM((M, _K), jnp.float32),
            pltpu.VMEM((M, _K), jnp.int32),
        ],
    )(query, ref)
    return d, i


def kernel(ref, query):
    d, i = _knn_topk(ref, query, C=2560, prec=jax.lax.Precision.DEFAULT)
    return d, i.astype(jnp.int64)


# speedup vs baseline: 1.7649x; 1.7649x over previous
import functools

import jax
import jax.numpy as jnp
from jax.experimental import pallas as pl
from jax.experimental.pallas import tpu as pltpu

_K = 16
_PAD_VAL = 1.0e18
_BIG_IDX = 2**30


def _knn_kernel(q_ref, r_ref, d_out, i_out, best_d, best_i, *, C, NC, prec):
    nc = pl.program_id(1)

    q = q_ref[0]
    r = r_ref[0]
    q2 = jnp.sum(q * q, axis=1)
    r2 = jnp.sum(r * r, axis=1)
    qr = jax.lax.dot_general(
        q, r, (((1,), (1,)), ((), ())),
        preferred_element_type=jnp.float32, precision=prec)
    d2 = q2[:, None] + r2[None, :] - 2.0 * qr
    d2 = jnp.maximum(d2, 0.0)

    M = q.shape[0]
    gidx = nc * C + jax.lax.broadcasted_iota(jnp.int32, (M, C), 1)

    @pl.when(nc == 0)
    def _init():
        best_d[...] = jnp.full((M, _K), jnp.inf, jnp.float32)
        best_i[...] = _BIG_IDX + jax.lax.broadcasted_iota(jnp.int32, (M, _K), 1)

    v = jnp.concatenate([best_d[...], d2], axis=1)
    g = jnp.concatenate([best_i[...], gidx], axis=1)

    mins, sels = [], []
    for _ in range(_K):
        mn = jnp.min(v, axis=1, keepdims=True)
        cand = jnp.where(v == mn, g, 2**31 - 1)
        sel = jnp.min(cand, axis=1, keepdims=True)
        mins.append(mn)
        sels.append(sel)
        v = jnp.where(g == sel, jnp.inf, v)
    best_d[...] = jnp.concatenate(mins, axis=1)
    best_i[...] = jnp.concatenate(sels, axis=1)

    @pl.when(nc == NC - 1)
    def _fin():
        d_out[0] = jnp.sqrt(best_d[...])
        i_out[0] = best_i[...]


def _knn_topk(ref, query, C, prec):
    B, N, D = ref.shape
    M = query.shape[1]
    NC = -(-N // C)
    Npad = NC * C
    if Npad != N:
        ref = jnp.pad(ref, ((0, 0), (0, Npad - N), (0, 0)),
                      constant_values=_PAD_VAL)

    grid = (B, NC)
    kern = functools.partial(_knn_kernel, C=C, NC=NC, prec=prec)
    d, i = pl.pallas_call(
        kern,
        grid=grid,
        in_specs=[
            pl.BlockSpec((1, M, D), lambda b, n: (b, 0, 0)),
            pl.BlockSpec((1, C, D), lambda b, n: (b, n, 0)),
        ],
        out_specs=[
            pl.BlockSpec((1, M, _K), lambda b, n: (b, 0, 0)),
            pl.BlockSpec((1, M, _K), lambda b, n: (b, 0, 0)),
        ],
        out_shape=[
            jax.ShapeDtypeStruct((B, M, _K), jnp.float32),
            jax.ShapeDtypeStruct((B, M, _K), jnp.int32),
        ],
        scratch_shapes=[
            pltpu.VMEM((M, _K), jnp.float32),
            pltpu.VMEM((M, _K), jnp.int32),
        ],
    )(query, ref)
    return d, i


def kernel(ref, query):
    d, i = _knn_topk(ref, query, C=2560, prec=jax.lax.Precision.DEFAULT)
    return d, i.astype(jnp.int64)


# speedup vs baseline: 2.9830x; 1.6902x over previous
import functools

import jax
import jax.numpy as jnp
from jax import lax
from jax.experimental import pallas as pl
from jax.experimental.pallas import tpu as pltpu
from jax.experimental.pallas import tpu_sc as plsc

_K = 16
_S = 18
_GRP = 128
_C = 2560
_PAD_VAL = 1.0e18
_BIG_IDX = 2**30
_NW = 32


def _p1_kernel(q_ref, r_ref, gsel_out, rsq_out, q2_out, amin, *, NC, G):
    nc = pl.program_id(1)
    q = q_ref[0]
    r = r_ref[0]
    M = q.shape[0]
    q2 = jnp.sum(q * q, axis=1)
    r2 = jnp.sum(r * r, axis=1)
    qr = lax.dot_general(q, r, (((1,), (1,)), ((), ())),
                         preferred_element_type=jnp.float32)
    d2 = jnp.maximum(q2[:, None] + r2[None, :] - 2.0 * qr, 0.0)
    GC = _C // _GRP
    mins = [jnp.min(d2[:, g * _GRP:(g + 1) * _GRP], axis=1, keepdims=True)
            for g in range(GC)]
    amin[nc] = jnp.concatenate(mins, axis=1)
    rsq_out[0, 0, 0] = r2
    q2_out[0, 0] = q2

    @pl.when(nc == NC - 1)
    def _fin():
        v = jnp.concatenate([amin[i] for i in range(NC)], axis=1)
        g = lax.broadcasted_iota(jnp.int32, (M, G), 1)
        sels = []
        for _ in range(_S):
            mn = jnp.min(v, axis=1, keepdims=True)
            cand = jnp.where(v == mn, g, 2**31 - 1)
            sel = jnp.min(cand, axis=1, keepdims=True)
            sels.append(sel)
            v = jnp.where(g == sel, jnp.inf, v)
        gsel_out[0] = jnp.concatenate(sels, axis=1)


def _phase1(refp, query):
    B, Npad, D = refp.shape
    M = query.shape[1]
    NC = Npad // _C
    G = Npad // _GRP
    return pl.pallas_call(
        functools.partial(_p1_kernel, NC=NC, G=G),
        grid=(B, NC),
        in_specs=[
            pl.BlockSpec((1, M, D), lambda b, n: (b, 0, 0)),
            pl.BlockSpec((1, _C, D), lambda b, n: (b, n, 0)),
        ],
        out_specs=[
            pl.BlockSpec((1, M, _S), lambda b, n: (b, 0, 0)),
            pl.BlockSpec((1, 1, 1, _C), lambda b, n: (b, n, 0, 0)),
            pl.BlockSpec((1, 1, M), lambda b, n: (b, 0, 0)),
        ],
        out_shape=[
            jax.ShapeDtypeStruct((B, M, _S), jnp.int32),
            jax.ShapeDtypeStruct((B, NC, 1, _C), jnp.float32),
            jax.ShapeDtypeStruct((B, 1, M), jnp.float32),
        ],
        scratch_shapes=[pltpu.VMEM((NC, M, _C // _GRP), jnp.float32)],
    )(query, refp)


def _round_bf16(x):
    b = lax.bitcast_convert_type(x, jnp.uint32)
    lsb = (b >> 16) & jnp.uint32(1)
    b = (b + jnp.uint32(0x7FFF) + lsb) & jnp.uint32(0xFFFF0000)
    return lax.bitcast_convert_type(b, jnp.float32)


def _lt_lex(d1, i1, d2, i2):
    return (d1 < d2) | ((d1 == d2) & (i1 < i2))


_GDN = lax.GatherDimensionNumbers(
    offset_dims=(), collapsed_slice_dims=(0,), start_index_map=(0,))


def _perm(x, idx):
    return lax.gather(x, idx[:, None], _GDN, (1,),
                      mode=lax.GatherScatterMode.PROMISE_IN_BOUNDS)


def _sc_topk(rblk, rsqt, idx3, idxb, qsplat, BM):
    RPW = BM // _NW
    NB = _S * 8
    mesh = plsc.VectorSubcoreMesh(core_axis_name="c", subcore_axis_name="s")

    @functools.partial(
        pl.kernel, mesh=mesh,
        compiler_params=pltpu.CompilerParams(needs_layout_passes=False,
                                             use_tc_tiling_on_sc=False),
        out_type=[jax.ShapeDtypeStruct((BM, _K), jnp.float32),
                  jax.ShapeDtypeStruct((BM, _K), jnp.int32)],
        scratch_types=[
            pltpu.VMEM((_S, _GRP), jnp.int32),
            pltpu.VMEM((2, NB // 2), jnp.int32),
            pltpu.VMEM((_S * _GRP, 16), jnp.float32),
            pltpu.VMEM((NB, 16), jnp.float32),
            pltpu.VMEM((17, 16), jnp.float32),
            pltpu.VMEM((1, _K), jnp.float32),
            pltpu.VMEM((1, _K), jnp.int32),
            pltpu.SemaphoreType.DMA,
        ])
    def k(rblk_h, rsq_h, idx_h, idxb_h, qs_h, d2_out, i_out,
          idxv, idxbv, V, Vr, Q, odb, oib, sem):
        wid = lax.axis_index("s") * 2 + lax.axis_index("c")
        iot = lax.iota(jnp.int32, 16)
        podd = jnp.clip(iot + jnp.where((iot & 1) == 1, 1, -1), 0, 15)

        def merge(args):
            bd, bi, dv, pv = args
            dk, pk = plsc.sort_key_val(dv, pv)
            rd = lax.rev(dk, (0,))
            ri = lax.rev(pk, (0,))
            m = _lt_lex(rd, ri, bd, bi)
            lod = jnp.where(m, rd, bd)
            loi = jnp.where(m, ri, bi)
            for dist in (8, 4, 2, 1):
                perm = iot ^ dist
                pd = _perm(lod, perm)
                pi = _perm(loi, perm)
                up = (iot & dist) == 0
                less = _lt_lex(pd, pi, lod, loi)
                take = jnp.where(up, less, ~less)
                lod = jnp.where(take, pd, lod)
                loi = jnp.where(take, pi, loi)
            return lod, loi

        def keep(args):
            return args[0], args[1]

        def blk(c, carry):
            bd, bi = carry
            s = c // 8
            j16 = (c - s * 8) * 16
            base = c * 16
            acc = jnp.zeros((16,), jnp.float32)
            for dd in range(16):
                acc = acc + V[base + dd] * Q[dd]
            dv = jnp.maximum(Q[16] + Vr[c] - 2.0 * acc, 0.0)
            pv = idxv[s, pl.ds(j16, 16)]
            bmaxv = _perm(bd, jnp.full((16,), 15, jnp.int32))
            imp = plsc.all_reduce_population_count(dv <= bmaxv)[0] > 0
            return lax.cond(imp, merge, keep, (bd, bi, dv, pv))

        def oddeven(bd, bi, perm, up):
            pd = _perm(bd, perm)
            pi = _perm(bi, perm)
            less = _lt_lex(pd, pi, bd, bi)
            take = jnp.where(up, less, ~less)
            return jnp.where(take, pd, bd), jnp.where(take, pi, bi)

        def row_body(r, _):
            row = wid * RPW + r
            pltpu.sync_copy(idx_h.at[row], idxv)
            pltpu.sync_copy(idxb_h.at[row], idxbv)
            pltpu.sync_copy(qs_h.at[pl.ds(row * 17, 17)], Q)
            cps = [pltpu.async_copy(rblk_h.at[idxv.at[s]],
                                    V.at[pl.ds(s * _GRP, _GRP)], sem)
                   for s in range(_S)]
            cps += [pltpu.async_copy(rsq_h.at[idxbv.at[t]],
                                     Vr.at[pl.ds(t * (NB // 2), NB // 2)],
                                     sem)
                    for t in range(2)]
            for cp in cps:
                cp.wait()
            bd0 = jnp.full((16,), jnp.inf, jnp.float32)
            bi0 = _BIG_IDX + iot
            bd, bi = lax.fori_loop(0, NB, blk, (bd0, bi0))
            bd, bi = oddeven(bd, bi, iot ^ 1, (iot & 1) == 0)
            bd, bi = oddeven(bd, bi, podd, (iot & 1) == 1)
            odb[0] = bd
            oib[0] = bi
            pltpu.sync_copy(odb, d2_out.at[pl.ds(row, 1)])
            pltpu.sync_copy(oib, i_out.at[pl.ds(row, 1)])
            return 0

        lax.fori_loop(0, RPW, row_body, 0)

    return k(rblk, rsqt, idx3, idxb, qsplat)


def kernel(ref, query):
    B, N, D = ref.shape
    M = query.shape[1]
    NC = -(-N // _C)
    Npad = NC * _C
    BM = B * M
    refp = jnp.pad(ref, ((0, 0), (0, Npad - N), (0, 0)),
                   constant_values=_PAD_VAL)

    gsel, rsq4, q2o = _phase1(refp, query)
    rsqt = rsq4.reshape(B * Npad // 16, 16)
    q2 = q2o.reshape(B, M)

    boffs = (jnp.arange(B, dtype=jnp.int32) * Npad)[:, None, None, None]
    idx3 = (gsel[..., None] * _GRP
            + jnp.arange(_GRP, dtype=jnp.int32) + boffs)
    idx3 = idx3.reshape(BM, _S, _GRP)
    idxb = (gsel[..., None] * 8 + jnp.arange(8, dtype=jnp.int32)
            + boffs // 16).reshape(BM, 2, _S * 8 // 2)

    rblk = refp.reshape(B, Npad // 16, 16, 16).swapaxes(2, 3)
    rblk = rblk.reshape(B * Npad, 16)
    rblk = _round_bf16(rblk)

    qb = _round_bf16(query)
    qsplat = jnp.concatenate([
        jnp.broadcast_to(qb[..., None], (B, M, 16, 16)),
        jnp.broadcast_to(q2[..., None, None], (B, M, 1, 16)),
    ], axis=2).reshape(BM * 17, 16)

    d2, i = _sc_topk(rblk, rsqt, idx3, idxb, qsplat, BM)

    d = jnp.sqrt(d2).reshape(B, M, _K)
    iout = (i.reshape(B, M, _K)
            - (jnp.arange(B, dtype=jnp.int32) * Npad)[:, None, None])
    return d, iout.astype(jnp.int64)


# speedup vs baseline: 3.1021x; 1.0399x over previous
import functools

import jax
import jax.numpy as jnp
from jax import lax
from jax.experimental import pallas as pl
from jax.experimental.pallas import tpu as pltpu
from jax.experimental.pallas import tpu_sc as plsc

_K = 16
_S = 18
_GRP = 128
_C = 2560
_PAD_VAL = 1.0e18
_BIG_IDX = 2**30
_NW = 32


def _p1_kernel(q_ref, r_ref, gsel_out, rsq_out, q2_out, amin, *, NC, G):
    nc = pl.program_id(1)
    q = q_ref[0]
    r = r_ref[0]
    M = q.shape[0]
    q2 = jnp.sum(q * q, axis=1)
    r2 = jnp.sum(r * r, axis=1)
    qr = lax.dot_general(q, r, (((1,), (1,)), ((), ())),
                         preferred_element_type=jnp.float32)
    d2 = jnp.maximum(q2[:, None] + r2[None, :] - 2.0 * qr, 0.0)
    GC = _C // _GRP
    mins = [jnp.min(d2[:, g * _GRP:(g + 1) * _GRP], axis=1, keepdims=True)
            for g in range(GC)]
    amin[nc] = jnp.concatenate(mins, axis=1)
    rsq_out[0, 0, 0] = r2
    q2_out[0, 0] = q2

    @pl.when(nc == NC - 1)
    def _fin():
        v = jnp.concatenate([amin[i] for i in range(NC)], axis=1)
        g = lax.broadcasted_iota(jnp.int32, (M, G), 1)
        sels = []
        for _ in range(_S):
            mn = jnp.min(v, axis=1, keepdims=True)
            cand = jnp.where(v == mn, g, 2**31 - 1)
            sel = jnp.min(cand, axis=1, keepdims=True)
            sels.append(sel)
            v = jnp.where(g == sel, jnp.inf, v)
        gsel_out[0] = jnp.concatenate(sels, axis=1)


def _phase1(refp, query):
    B, Npad, D = refp.shape
    M = query.shape[1]
    NC = Npad // _C
    G = Npad // _GRP
    return pl.pallas_call(
        functools.partial(_p1_kernel, NC=NC, G=G),
        grid=(B, NC),
        in_specs=[
            pl.BlockSpec((1, M, D), lambda b, n: (b, 0, 0)),
            pl.BlockSpec((1, _C, D), lambda b, n: (b, n, 0)),
        ],
        out_specs=[
            pl.BlockSpec((1, M, _S), lambda b, n: (b, 0, 0)),
            pl.BlockSpec((1, 1, 1, _C), lambda b, n: (b, n, 0, 0)),
            pl.BlockSpec((1, 1, M), lambda b, n: (b, 0, 0)),
        ],
        out_shape=[
            jax.ShapeDtypeStruct((B, M, _S), jnp.int32),
            jax.ShapeDtypeStruct((B, NC, 1, _C), jnp.float32),
            jax.ShapeDtypeStruct((B, 1, M), jnp.float32),
        ],
        scratch_shapes=[pltpu.VMEM((NC, M, _C // _GRP), jnp.float32)],
    )(query, refp)


def _round_bf16(x):
    b = lax.bitcast_convert_type(x, jnp.uint32)
    lsb = (b >> 16) & jnp.uint32(1)
    b = (b + jnp.uint32(0x7FFF) + lsb) & jnp.uint32(0xFFFF0000)
    return lax.bitcast_convert_type(b, jnp.float32)


def _lt_lex(d1, i1, d2, i2):
    return (d1 < d2) | ((d1 == d2) & (i1 < i2))


_GDN = lax.GatherDimensionNumbers(
    offset_dims=(), collapsed_slice_dims=(0,), start_index_map=(0,))


def _perm(x, idx):
    return lax.gather(x, idx[:, None], _GDN, (1,),
                      mode=lax.GatherScatterMode.PROMISE_IN_BOUNDS)


_MLEN = _S * _GRP + _S * 8 + 17 * 16


def _sc_topk(rblk, rsqt, meta, BM):
    RPW = BM // _NW
    NB = _S * 8
    OQ = _S * _GRP + NB
    mesh = plsc.VectorSubcoreMesh(core_axis_name="c", subcore_axis_name="s")

    @functools.partial(
        pl.kernel, mesh=mesh,
        compiler_params=pltpu.CompilerParams(needs_layout_passes=False,
                                             use_tc_tiling_on_sc=False),
        out_type=jax.ShapeDtypeStruct((BM, 2 * _K), jnp.int32),
        scratch_types=[
            pltpu.VMEM((_MLEN,), jnp.int32),
            pltpu.VMEM((_S * _GRP, 16), jnp.float32),
            pltpu.VMEM((NB, 16), jnp.float32),
            pltpu.VMEM((1, 2 * _K), jnp.int32),
            pltpu.SemaphoreType.DMA,
        ])
    def k(rblk_h, rsq_h, meta_h, out_h, mv, V, Vr, ob, sem):
        wid = lax.axis_index("s") * 2 + lax.axis_index("c")
        iot = lax.iota(jnp.int32, 16)
        podd = jnp.clip(iot + jnp.where((iot & 1) == 1, 1, -1), 0, 15)

        def merge(args):
            bd, bi, dv, pv = args
            dk, pk = plsc.sort_key_val(dv, pv)
            rd = lax.rev(dk, (0,))
            ri = lax.rev(pk, (0,))
            m = _lt_lex(rd, ri, bd, bi)
            lod = jnp.where(m, rd, bd)
            loi = jnp.where(m, ri, bi)
            for dist in (8, 4, 2, 1):
                perm = iot ^ dist
                pd = _perm(lod, perm)
                pi = _perm(loi, perm)
                up = (iot & dist) == 0
                less = _lt_lex(pd, pi, lod, loi)
                take = jnp.where(up, less, ~less)
                lod = jnp.where(take, pd, lod)
                loi = jnp.where(take, pi, loi)
            return lod, loi

        def keep(args):
            return args[0], args[1]

        def blk(c, carry):
            bd, bi = carry
            base = c * 16
            qs = [lax.bitcast_convert_type(mv[pl.ds(OQ + dd * 16, 16)],
                                           jnp.float32) for dd in range(17)]
            acc = jnp.zeros((16,), jnp.float32)
            for dd in range(16):
                acc = acc + V[base + dd] * qs[dd]
            dv = jnp.maximum(qs[16] + Vr[c] - 2.0 * acc, 0.0)
            pv = mv[pl.ds(base, 16)]
            bmaxv = _perm(bd, jnp.full((16,), 15, jnp.int32))
            imp = plsc.all_reduce_population_count(dv <= bmaxv)[0] > 0
            return lax.cond(imp, merge, keep, (bd, bi, dv, pv))

        def oddeven(bd, bi, perm, up):
            pd = _perm(bd, perm)
            pi = _perm(bi, perm)
            less = _lt_lex(pd, pi, bd, bi)
            take = jnp.where(up, less, ~less)
            return jnp.where(take, pd, bd), jnp.where(take, pi, bi)

        def row_body(r, _):
            row = wid * RPW + r
            pltpu.sync_copy(meta_h.at[row], mv)
            cps = [pltpu.async_copy(rblk_h.at[mv.at[pl.ds(s * _GRP, _GRP)]],
                                    V.at[pl.ds(s * _GRP, _GRP)], sem)
                   for s in range(_S)]
            cps += [pltpu.async_copy(
                        rsq_h.at[mv.at[pl.ds(_S * _GRP + t * (NB // 2),
                                             NB // 2)]],
                        Vr.at[pl.ds(t * (NB // 2), NB // 2)], sem)
                    for t in range(2)]
            for cp in cps:
                cp.wait()
            bd0 = jnp.full((16,), jnp.inf, jnp.float32)
            bi0 = _BIG_IDX + iot
            bd, bi = lax.fori_loop(0, NB, blk, (bd0, bi0))
            bd, bi = oddeven(bd, bi, iot ^ 1, (iot & 1) == 0)
            bd, bi = oddeven(bd, bi, podd, (iot & 1) == 1)
            ob[0, pl.ds(0, 16)] = lax.bitcast_convert_type(bd, jnp.int32)
            ob[0, pl.ds(16, 16)] = bi
            pltpu.sync_copy(ob, out_h.at[pl.ds(row, 1)])
            return 0

        lax.fori_loop(0, RPW, row_body, 0)

    return k(rblk, rsqt, meta)


def kernel(ref, query):
    B, N, D = ref.shape
    M = query.shape[1]
    NC = -(-N // _C)
    Npad = NC * _C
    BM = B * M
    refp = jnp.pad(ref, ((0, 0), (0, Npad - N), (0, 0)),
                   constant_values=_PAD_VAL)

    gsel, rsq4, q2o = _phase1(refp, query)
    rsqt = rsq4.reshape(B * Npad // 16, 16)
    q2 = q2o.reshape(B, M)

    boffs = (jnp.arange(B, dtype=jnp.int32) * Npad)[:, None, None, None]
    idx3 = (gsel[..., None] * _GRP
            + jnp.arange(_GRP, dtype=jnp.int32) + boffs)
    idx3 = idx3.reshape(BM, _S * _GRP)
    idxb = (gsel[..., None] * 8 + jnp.arange(8, dtype=jnp.int32)
            + boffs // 16).reshape(BM, _S * 8)

    rblk = refp.reshape(B, Npad // 16, 16, 16).swapaxes(2, 3)
    rblk = rblk.reshape(B * Npad, 16)
    rblk = _round_bf16(rblk)

    qb = _round_bf16(query)
    qsplat = jnp.concatenate([
        jnp.broadcast_to(qb[..., None], (B, M, 16, 16)),
        jnp.broadcast_to(q2[..., None, None], (B, M, 1, 16)),
    ], axis=2).reshape(BM, 17 * 16)

    meta = jnp.concatenate(
        [idx3, idxb, lax.bitcast_convert_type(qsplat, jnp.int32)], axis=1)

    out = _sc_topk(rblk, rsqt, meta, BM)

    d2 = lax.bitcast_convert_type(out[:, :_K], jnp.float32)
    d = jnp.sqrt(d2).reshape(B, M, _K)
    iout = (out[:, _K:].reshape(B, M, _K)
            - (jnp.arange(B, dtype=jnp.int32) * Npad)[:, None, None])
    return d, iout.astype(jnp.int64)


# speedup vs baseline: 3.6568x; 1.1788x over previous
import functools

import jax
import jax.numpy as jnp
from jax import lax
from jax.experimental import pallas as pl
from jax.experimental.pallas import tpu as pltpu
from jax.experimental.pallas import tpu_sc as plsc

_K = 16
_S = 18
_GRP = 128
_C = 2560
_PAD_VAL = 1.0e18
_BIG_IDX = 2**30
_NW = 32


def _p1_kernel(q_ref, r_ref, gsel_out, rsq_out, q2_out, amin, *, NC, G):
    nc = pl.program_id(1)
    q = q_ref[0]
    r = r_ref[0]
    M = q.shape[0]
    q2 = jnp.sum(q * q, axis=1)
    r2 = jnp.sum(r * r, axis=1)
    qr = lax.dot_general(q, r, (((1,), (1,)), ((), ())),
                         preferred_element_type=jnp.float32)
    d2 = jnp.maximum(q2[:, None] + r2[None, :] - 2.0 * qr, 0.0)
    GC = _C // _GRP
    mins = [jnp.min(d2[:, g * _GRP:(g + 1) * _GRP], axis=1, keepdims=True)
            for g in range(GC)]
    amin[nc] = jnp.concatenate(mins, axis=1)
    rsq_out[0, 0, 0] = r2
    q2_out[0, 0] = q2

    @pl.when(nc == NC - 1)
    def _fin():
        v = jnp.concatenate([amin[i] for i in range(NC)], axis=1)
        g = lax.broadcasted_iota(jnp.int32, (M, G), 1)
        sels = []
        for _ in range(_S):
            mn = jnp.min(v, axis=1, keepdims=True)
            cand = jnp.where(v == mn, g, 2**31 - 1)
            sel = jnp.min(cand, axis=1, keepdims=True)
            sels.append(sel)
            v = jnp.where(g == sel, jnp.inf, v)
        gsel_out[0] = jnp.concatenate(sels, axis=1)


def _phase1(refp, query):
    B, Npad, D = refp.shape
    M = query.shape[1]
    NC = Npad // _C
    G = Npad // _GRP
    return pl.pallas_call(
        functools.partial(_p1_kernel, NC=NC, G=G),
        grid=(B, NC),
        in_specs=[
            pl.BlockSpec((1, M, D), lambda b, n: (b, 0, 0)),
            pl.BlockSpec((1, _C, D), lambda b, n: (b, n, 0)),
        ],
        out_specs=[
            pl.BlockSpec((1, M, _S), lambda b, n: (b, 0, 0)),
            pl.BlockSpec((1, 1, 1, _C), lambda b, n: (b, n, 0, 0)),
            pl.BlockSpec((1, 1, M), lambda b, n: (b, 0, 0)),
        ],
        out_shape=[
            jax.ShapeDtypeStruct((B, M, _S), jnp.int32),
            jax.ShapeDtypeStruct((B, NC, 1, _C), jnp.float32),
            jax.ShapeDtypeStruct((B, 1, M), jnp.float32),
        ],
        scratch_shapes=[pltpu.VMEM((NC, M, _C // _GRP), jnp.float32)],
    )(query, refp)


def _round_bf16(x):
    b = lax.bitcast_convert_type(x, jnp.uint32)
    lsb = (b >> 16) & jnp.uint32(1)
    b = (b + jnp.uint32(0x7FFF) + lsb) & jnp.uint32(0xFFFF0000)
    return lax.bitcast_convert_type(b, jnp.float32)


def _lt_lex(d1, i1, d2, i2):
    return (d1 < d2) | ((d1 == d2) & (i1 < i2))


_GDN = lax.GatherDimensionNumbers(
    offset_dims=(), collapsed_slice_dims=(0,), start_index_map=(0,))


def _perm(x, idx):
    return lax.gather(x, idx[:, None], _GDN, (1,),
                      mode=lax.GatherScatterMode.PROMISE_IN_BOUNDS)


_MLEN = _S * _GRP + _S * 8 + 17 * 16


def _sc_topk(rblk, rsqt, meta, BM):
    RPW = BM // _NW
    NB = _S * 8
    OQ = _S * _GRP + NB
    mesh = plsc.VectorSubcoreMesh(core_axis_name="c", subcore_axis_name="s")

    @functools.partial(
        pl.kernel, mesh=mesh,
        compiler_params=pltpu.CompilerParams(needs_layout_passes=False,
                                             use_tc_tiling_on_sc=False),
        out_type=jax.ShapeDtypeStruct((BM, 2 * _K), jnp.int32),
        scratch_types=[
            pltpu.VMEM((2, _MLEN), jnp.int32),
            pltpu.VMEM((2, _S * _GRP, 16), jnp.float32),
            pltpu.VMEM((2, NB, 16), jnp.float32),
            pltpu.VMEM((1, 2 * _K), jnp.int32),
            pltpu.SemaphoreType.DMA,
            pltpu.SemaphoreType.DMA,
        ])
    def k(rblk_h, rsq_h, meta_h, out_h, mv2, V2, Vr2, ob, semg0, semg1):
        wid = lax.axis_index("s") * 2 + lax.axis_index("c")
        iot = lax.iota(jnp.int32, 16)
        podd = jnp.clip(iot + jnp.where((iot & 1) == 1, 1, -1), 0, 15)

        def merge(args):
            bd, bi, dv, pv = args
            dk, pk = plsc.sort_key_val(dv, pv)
            rd = lax.rev(dk, (0,))
            ri = lax.rev(pk, (0,))
            m = _lt_lex(rd, ri, bd, bi)
            lod = jnp.where(m, rd, bd)
            loi = jnp.where(m, ri, bi)
            for dist in (8, 4, 2, 1):
                perm = iot ^ dist
                pd = _perm(lod, perm)
                pi = _perm(loi, perm)
                up = (iot & dist) == 0
                less = _lt_lex(pd, pi, lod, loi)
                take = jnp.where(up, less, ~less)
                lod = jnp.where(take, pd, lod)
                loi = jnp.where(take, pi, loi)
            return lod, loi

        def keep(args):
            return args[0], args[1]

        semg = (semg0, semg1)

        def mk_blk(p):
            def blk(c, carry):
                bd, bi = carry
                base = c * 16
                qs = [lax.bitcast_convert_type(
                          mv2[p, pl.ds(OQ + dd * 16, 16)], jnp.float32)
                      for dd in range(17)]
                acc = jnp.zeros((16,), jnp.float32)
                for dd in range(16):
                    acc = acc + V2[p, base + dd] * qs[dd]
                dv = jnp.maximum(qs[16] + Vr2[p, c] - 2.0 * acc, 0.0)
                pv = mv2[p, pl.ds(base, 16)]
                bmaxv = _perm(bd, jnp.full((16,), 15, jnp.int32))
                imp = plsc.all_reduce_population_count(dv <= bmaxv)[0] > 0
                return lax.cond(imp, merge, keep, (bd, bi, dv, pv))
            return blk

        def _gather_cps(p):
            cps = [pltpu.make_async_copy(
                       rblk_h.at[mv2.at[p, pl.ds(s * _GRP, _GRP)]],
                       V2.at[p, pl.ds(s * _GRP, _GRP)], semg[p])
                   for s in range(_S)]
            cps += [pltpu.make_async_copy(
                        rsq_h.at[mv2.at[p, pl.ds(_S * _GRP + t * (NB // 2),
                                                 NB // 2)]],
                        Vr2.at[p, pl.ds(t * (NB // 2), NB // 2)], semg[p])
                    for t in range(2)]
            return cps

        def fire(row, p):
            pltpu.sync_copy(meta_h.at[row], mv2.at[p])
            for cp in _gather_cps(p):
                cp.start()

        def waitg(p):
            for cp in _gather_cps(p):
                cp.wait()

        def oddeven(bd, bi, perm, up):
            pd = _perm(bd, perm)
            pi = _perm(bi, perm)
            less = _lt_lex(pd, pi, bd, bi)
            take = jnp.where(up, less, ~less)
            return jnp.where(take, pd, bd), jnp.where(take, pi, bi)

        rbase = wid * RPW

        def pair_body(g, _):
            r0 = rbase + 2 * g
            for p in (0, 1):
                row = r0 + p

                @pl.when(row + 1 < rbase + RPW)
                def _pf():
                    fire(row + 1, 1 - p)

                waitg(p)
                bd0 = jnp.full((16,), jnp.inf, jnp.float32)
                bi0 = _BIG_IDX + iot
                bd, bi = lax.fori_loop(0, NB, mk_blk(p), (bd0, bi0))
                bd, bi = oddeven(bd, bi, iot ^ 1, (iot & 1) == 0)
                bd, bi = oddeven(bd, bi, podd, (iot & 1) == 1)
                ob[0, pl.ds(0, 16)] = lax.bitcast_convert_type(bd, jnp.int32)
                ob[0, pl.ds(16, 16)] = bi
                pltpu.sync_copy(ob, out_h.at[pl.ds(row, 1)])
            return 0

        fire(rbase, 0)
        lax.fori_loop(0, RPW // 2, pair_body, 0)

    return k(rblk, rsqt, meta)


def kernel(ref, query):
    B, N, D = ref.shape
    M = query.shape[1]
    NC = -(-N // _C)
    Npad = NC * _C
    BM = B * M
    refp = jnp.pad(ref, ((0, 0), (0, Npad - N), (0, 0)),
                   constant_values=_PAD_VAL)

    gsel, rsq4, q2o = _phase1(refp, query)
    rsqt = rsq4.reshape(B * Npad // 16, 16)
    q2 = q2o.reshape(B, M)

    boffs = (jnp.arange(B, dtype=jnp.int32) * Npad)[:, None, None, None]
    idx3 = (gsel[..., None] * _GRP
            + jnp.arange(_GRP, dtype=jnp.int32) + boffs)
    idx3 = idx3.reshape(BM, _S * _GRP)
    idxb = (gsel[..., None] * 8 + jnp.arange(8, dtype=jnp.int32)
            + boffs // 16).reshape(BM, _S * 8)

    rblk = refp.reshape(B, Npad // 16, 16, 16).swapaxes(2, 3)
    rblk = rblk.reshape(B * Npad, 16)
    rblk = _round_bf16(rblk)

    qb = _round_bf16(query)
    qsplat = jnp.concatenate([
        jnp.broadcast_to(qb[..., None], (B, M, 16, 16)),
        jnp.broadcast_to(q2[..., None, None], (B, M, 1, 16)),
    ], axis=2).reshape(BM, 17 * 16)

    meta = jnp.concatenate(
        [idx3, idxb, lax.bitcast_convert_type(qsplat, jnp.int32)], axis=1)

    out = _sc_topk(rblk, rsqt, meta, BM)

    d2 = lax.bitcast_convert_type(out[:, :_K], jnp.float32)
    d = jnp.sqrt(d2).reshape(B, M, _K)
    iout = (out[:, _K:].reshape(B, M, _K)
            - (jnp.arange(B, dtype=jnp.int32) * Npad)[:, None, None])
    return d, iout.astype(jnp.int64)
